# baseline (speedup 1.0000x reference)
import functools

import jax
import jax.numpy as jnp
from jax import lax
from jax.experimental import pallas as pl
from jax.experimental.pallas import tpu as pltpu
from jax.experimental.pallas import tpu_sc as plsc

G = 128
NK = 27
KC = 28
CIN = 64
COUT = 64
WC = KC * COUT
TBL = G * G * G
TBLP = TBL + 128
NC = 2
NS = 16
NW = NC * NS

_OFFS = [(dz, dy, dx) for dz in (-1, 0, 1) for dy in (-1, 0, 1)
         for dx in (-1, 0, 1)]

_mesh = functools.partial(plsc.VectorSubcoreMesh, core_axis_name="c",
                          subcore_axis_name="s", num_cores=NC,
                          num_subcores=NS)


def _gemm(feats_pad, w_all, npad):
    bm = 512

    def body(f_ref, w_ref, o_ref):
        o_ref[...] = jnp.dot(f_ref[...], w_ref[...],
                             preferred_element_type=jnp.float32
                             ).reshape(1, bm, 128)

    return pl.pallas_call(
        body,
        grid=(npad // bm, KC // 2),
        in_specs=[pl.BlockSpec((bm, CIN), lambda i, m: (i, 0)),
                  pl.BlockSpec((CIN, 128), lambda i, m: (0, m))],
        out_specs=pl.BlockSpec((1, bm, 128), lambda i, m: (m, i, 0)),
        out_shape=jax.ShapeDtypeStruct((KC // 2, npad, 128), jnp.float32),
    )(feats_pad, w_all)


def _build_table(cz, cy, cx, npad):
    ib = 8192
    b2 = 64
    ppt = npad // NS
    nb = ppt // b2

    @functools.partial(
        pl.kernel,
        out_type=jax.ShapeDtypeStruct((TBLP,), jnp.int32),
        mesh=_mesh(),
        scratch_types=[
            pltpu.VMEM((ib,), jnp.int32),
            pltpu.VMEM((b2,), jnp.int32),
            pltpu.VMEM((b2,), jnp.int32),
            pltpu.VMEM((b2,), jnp.int32),
            pltpu.VMEM((b2,), jnp.int32),
            pltpu.VMEM((b2,), jnp.int32),
            pltpu.VMEM((b2,), jnp.int32),
            pltpu.SemaphoreType.DMA,
        ],
    )
    def k(czr, cyr, cxr, tbl, ibuf, czv, cyv, cxv, keyv, idsv, gotv, sem):
        cid = lax.axis_index("c")
        tid = lax.axis_index("s")
        lane = lax.iota(jnp.int32, 16)

        @pl.when(cid == 0)
        def _():
            words = TBL // NS

            def fill(i, _):
                ibuf[pl.ds(i * 16, 16)] = jnp.full((16,), -1, jnp.int32)
                return 0
            lax.fori_loop(0, ib // 16, fill, 0)

            def icopy(m, _):
                pltpu.sync_copy(ibuf, tbl.at[pl.ds(tid * words + m * ib, ib)])
                return 0
            lax.fori_loop(0, words // ib, icopy, 0)

            @pl.when(tid == 0)
            def _():
                pltpu.sync_copy(ibuf.at[pl.ds(0, 128)],
                                tbl.at[pl.ds(TBL, 128)])

            plsc.subcore_barrier()

            def load_keys(b):
                base = tid * ppt + b * b2
                pltpu.sync_copy(czr.at[pl.ds(base, b2)], czv)
                pltpu.sync_copy(cyr.at[pl.ds(base, b2)], cyv)
                pltpu.sync_copy(cxr.at[pl.ds(base, b2)], cxv)

                def kcomp(j, _):
                    s = pl.ds(j * 16, 16)
                    z, y, x = czv[s], cyv[s], cxv[s]
                    key = (z * G + y) * G + x
                    dump = TBL + ((b * 4 + j) % 7) * 16 + lane
                    keyv[s] = jnp.where(z >= 0, key, dump)
                    idsv[s] = base + j * 16 + lane
                    return 0
                lax.fori_loop(0, b2 // 16, kcomp, 0)

            def scat(b, _):
                load_keys(b)
                pltpu.async_copy(idsv, tbl.at[keyv], sem).wait()
                return 0
            lax.fori_loop(0, nb, scat, 0)
            plsc.subcore_barrier()

            for _r in range(3):
                def corr(b, _):
                    load_keys(b)
                    pltpu.async_copy(tbl.at[keyv], gotv, sem).wait()

                    def qcomp(j, _):
                        s = pl.ds(j * 16, 16)
                        got, ids, key = gotv[s], idsv[s], keyv[s]
                        dump = TBL + 64 + ((b * 4 + j) % 3) * 16 + lane
                        qual = (ids > got) & (key < TBL)
                        keyv[s] = jnp.where(qual, key, dump)
                        return 0
                    lax.fori_loop(0, b2 // 16, qcomp, 0)
                    pltpu.async_copy(idsv, tbl.at[keyv], sem).wait()
                    return 0
                lax.fori_loop(0, nb, corr, 0)
                plsc.subcore_barrier()

    return k(cz, cy, cx)


def _probe(cz, cy, cx, table, n, npad):
    b3 = 112
    ppt = npad // NW
    nb = ppt // b3

    @functools.partial(
        pl.kernel,
        out_type=jax.ShapeDtypeStruct((NK * npad,), jnp.int32),
        mesh=_mesh(),
        scratch_types=[
            pltpu.VMEM((b3,), jnp.int32),
            pltpu.VMEM((b3,), jnp.int32),
            pltpu.VMEM((b3,), jnp.int32),
            pltpu.VMEM((NK, b3), jnp.int32),
            pltpu.VMEM((NK, b3), jnp.int32),
            pltpu.VMEM((NK, b3), jnp.int32),
            pltpu.SemaphoreType.DMA,
        ],
    )
    def k(czr, cyr, cxr, tbl, fidx, czv, cyv, cxv, kbuf, gbuf, fbuf, sem):
        cid = lax.axis_index("c")
        sid = lax.axis_index("s")
        wid = sid * NC + cid

        def batch(b, _):
            base = wid * ppt + b * b3
            pltpu.sync_copy(czr.at[pl.ds(base, b3)], czv)
            pltpu.sync_copy(cyr.at[pl.ds(base, b3)], cyv)
            pltpu.sync_copy(cxr.at[pl.ds(base, b3)], cxv)

            def kcomp(j, _):
                s = pl.ds(j * 16, 16)
                z, y, x = czv[s], cyv[s], cxv[s]
                key0 = (z * G + y) * G + x
                pz = z >= 0
                for kk, (dz, dy, dx) in enumerate(_OFFS):
                    z2, y2, x2 = z + dz, y + dy, x + dx
                    v = (pz & (z2 >= 0) & (z2 < G) & (y2 >= 0) & (y2 < G)
                         & (x2 >= 0) & (x2 < G))
                    nk = key0 + (dz * G + dy) * G + dx
                    kbuf[kk, s] = jnp.where(v, nk, TBL + 112 + ((kk + j) % 16))
                return 0
            lax.fori_loop(0, b3 // 16, kcomp, 0)

            cps = [pltpu.async_copy(tbl.at[kbuf.at[kk]], gbuf.at[kk], sem)
                   for kk in range(NK)]
            for c in cps:
                c.wait()

            def fcomp(j, _):
                s = pl.ds(j * 16, 16)
                for kk in range(NK):
                    got = gbuf[kk, s]
                    fbuf[kk, s] = ((kk // 2) * npad
                                   + jnp.where(got >= 0, got, n + kk))
                return 0
            lax.fori_loop(0, b3 // 16, fcomp, 0)

            for kk in range(NK):
                pltpu.sync_copy(fbuf.at[kk],
                                fidx.at[pl.ds(kk * npad + base, b3)])
            return 0
        lax.fori_loop(0, nb, batch, 0)

    return k(cz, cy, cx, table)


def _gather_accum(fidx, tmp_flat, npad):
    b4 = 32
    nka = 14
    ppt = npad // NW
    nb = ppt // b4

    @functools.partial(
        pl.kernel,
        out_type=jax.ShapeDtypeStruct((npad, COUT), jnp.float32),
        mesh=_mesh(),
        scratch_types=[
            pltpu.VMEM((NK, b4), jnp.int32),
            pltpu.VMEM((nka, b4, 128), jnp.float32),
            pltpu.VMEM((b4, COUT), jnp.float32),
            pltpu.SemaphoreType.DMA,
        ],
    )
    def k(fidxr, tflat, out, fbuf, gdst, obuf, sem):
        cid = lax.axis_index("c")
        sid = lax.axis_index("s")
        wid = sid * NC + cid

        def batch(b, _):
            base = wid * ppt + b * b4
            for kk in range(NK):
                pltpu.sync_copy(fidxr.at[pl.ds(kk * npad + base, b4)],
                                fbuf.at[kk])
            for lo in (0, nka):
                taps = range(lo, min(lo + nka, NK))
                cps = [pltpu.async_copy(tflat.at[fbuf.at[kk]],
                                        gdst.at[kk - lo], sem)
                       for kk in taps]
                for c in cps:
                    c.wait()

                def pcomp(p, _, lo=lo, taps=taps):
                    if lo == 0:
                        accs = [jnp.zeros((16,), jnp.float32)
                                for _ in range(4)]
                    else:
                        accs = [obuf[p, pl.ds(j * 16, 16)] for j in range(4)]
                    for kk in taps:
                        h = (kk % 2) * COUT
                        for j in range(4):
                            accs[j] = accs[j] + gdst[kk - lo, p,
                                                     pl.ds(h + j * 16, 16)]
                    for j in range(4):
                        obuf[p, pl.ds(j * 16, 16)] = accs[j]
                    return 0
                lax.fori_loop(0, b4, pcomp, 0)
            pltpu.sync_copy(obuf, out.at[pl.ds(base, b4)])
            return 0
        lax.fori_loop(0, nb, batch, 0)

    return k(fidx, tmp_flat)


def kernel(feats, coords, weight):
    n = feats.shape[0]
    npad = ((n + 1023) // 1024) * 1024
    feats_pad = jnp.pad(feats, ((0, npad - n), (0, 0)))
    w_all = jnp.pad(weight.astype(jnp.float32).transpose(1, 0, 2)
                    .reshape(CIN, NK * COUT), ((0, 0), (0, COUT)))
    cpad = jnp.pad(coords, ((0, npad - n), (0, 0)), constant_values=-1)
    cz, cy, cx = cpad[:, 0], cpad[:, 1], cpad[:, 2]

    tmp = _gemm(feats_pad, w_all, npad)
    tmp_flat = tmp.reshape(KC // 2 * npad, 128)
    table = _build_table(cz, cy, cx, npad)
    fidx = _probe(cz, cy, cx, table, n, npad)
    out = _gather_accum(fidx, tmp_flat, npad)
    return out[:n]
